# baseline (speedup 1.0000x reference)
import functools
import math

import jax
import jax.numpy as jnp
from jax import lax
from jax.experimental import pallas as pl
from jax.experimental.pallas import tpu as pltpu
from jax.experimental.pallas import tpu_sc as plsc

N_NODES = 10000
FEAT = 3072
NL = N_NODES * FEAT
TC_BLOCK = 40
TC_GRID = N_NODES // TC_BLOCK


def _conv2ch(x, w_ref, b_ref):
    B = x.shape[0]
    z = jnp.zeros((B, 3), dtype=jnp.float32)
    acc0 = jnp.full((B, FEAT), b_ref[0], dtype=jnp.float32)
    acc1 = jnp.full((B, FEAT), b_ref[1], dtype=jnp.float32)
    for i in range(2):
        xp = jnp.concatenate([z, x[:, i, :], z], axis=1)
        for k in range(7):
            s = xp[:, k:k + FEAT]
            acc0 = acc0 + w_ref[0, i, k] * s
            acc1 = acc1 + w_ref[1, i, k] * s
    return acc0, acc1


def _stats_update(i, h0, h1, st_ref, acc_ref):
    @pl.when(i == 0)
    def _():
        for j in range(4):
            acc_ref[j] = 0.0

    acc_ref[0] += jnp.sum(h0)
    acc_ref[1] += jnp.sum(h1)
    acc_ref[2] += jnp.sum(h0 * h0)
    acc_ref[3] += jnp.sum(h1 * h1)

    @pl.when(i == TC_GRID - 1)
    def _():
        for j in range(4):
            st_ref[j] = acc_ref[j]


def _p1_body(x_ref, w_ref, b_ref, h_ref, st_ref, acc_ref):
    i = pl.program_id(0)
    h0, h1 = _conv2ch(x_ref[...], w_ref, b_ref)
    h_ref[:, 0, :] = h0
    h_ref[:, 1, :] = h1
    _stats_update(i, h0, h1, st_ref, acc_ref)


def _p2_body(x_ref, w_ref, b_ref, A_ref, B_ref, h_ref, st_ref, acc_ref):
    i = pl.program_id(0)
    x = x_ref[...]
    g0 = jnp.maximum(x[:, 0, :] * A_ref[0] + B_ref[0], 0.0)
    g1 = jnp.maximum(x[:, 1, :] * A_ref[1] + B_ref[1], 0.0)
    g = jnp.stack([g0, g1], axis=1)
    h0, h1 = _conv2ch(g, w_ref, b_ref)
    h_ref[:, 0, :] = h0
    h_ref[:, 1, :] = h1
    _stats_update(i, h0, h1, st_ref, acc_ref)


def _conv_stats_pass(body, args, interpret=False):
    return pl.pallas_call(
        body,
        grid=(TC_GRID,),
        in_specs=[pl.BlockSpec((TC_BLOCK, 2, FEAT), lambda i: (i, 0, 0))]
        + [pl.BlockSpec(memory_space=pltpu.SMEM)] * (len(args) - 1),
        out_specs=[
            pl.BlockSpec((TC_BLOCK, 2, FEAT), lambda i: (i, 0, 0)),
            pl.BlockSpec(memory_space=pltpu.SMEM),
        ],
        out_shape=[
            jax.ShapeDtypeStruct((N_NODES, 2, FEAT), jnp.float32),
            jax.ShapeDtypeStruct((4,), jnp.float32),
        ],
        scratch_shapes=[pltpu.SMEM((4,), jnp.float32)],
        interpret=interpret,
    )(*args)


def _affine_from_stats(st, gamma, beta, eps=1e-5):
    mean = st[:2] / NL
    var = st[2:] / NL - mean * mean
    A = gamma / jnp.sqrt(var + eps)
    B = beta - mean * A
    return A, B


def _dense_part(x, conv1_w, conv1_b, bn1_g, bn1_b, conv2_w, conv2_b, bn2_g, bn2_b,
                interpret=False):
    h1, st1 = _conv_stats_pass(_p1_body, (x, conv1_w, conv1_b), interpret)
    A1, B1 = _affine_from_stats(st1, bn1_g, bn1_b)
    h2, st2 = _conv_stats_pass(_p2_body, (h1, conv2_w, conv2_b, A1, B1), interpret)
    A2, B2 = _affine_from_stats(st2, bn2_g, bn2_b)
    return h2, A2, B2


ROLL_BLK = 320
GRP = 8
N_PAD = 10240
WINW = 128
_WIN0 = 990
_WIN1 = 1490


def _sc_roll_body(h2, dtp, dts, coef, T, W, bigin, bigout,
                  wbuf, dtbuf, coefv, dbuf):
    cid = lax.axis_index("c")
    sid = lax.axis_index("s")
    wid = sid * 2 + cid
    base = wid * ROLL_BLK
    nrows = jnp.minimum(ROLL_BLK, N_NODES - base)
    ngrp = (nrows + GRP - 1) // GRP
    pltpu.sync_copy(dtp.at[pl.ds(base, ROLL_BLK)], dtbuf.at[pl.ds(0, ROLL_BLK)])
    pltpu.sync_copy(dts.at[pl.ds(base, ROLL_BLK)], dtbuf.at[pl.ds(ROLL_BLK, ROLL_BLK)])
    pltpu.sync_copy(coef, coefv)
    lane = lax.iota(jnp.int32, 16)
    cv = coefv[...]
    zf16 = jnp.zeros((16,), jnp.float32)
    for r0 in range(GRP):
        for k0 in range(WINW // 16):
            wbuf[r0, pl.ds(k0 * 16, 16)] = zf16

    def grp_body(g, _):
        n8 = base + g * GRP
        pltpu.sync_copy(h2.at[pl.ds(n8, GRP)], bigin)
        dtv0 = dtbuf[pl.ds(g * GRP, 16)]
        dtv1 = dtbuf[pl.ds(g * GRP + ROLL_BLK, 16)]
        for r in range(GRP):
            for c in range(2):
                sft = dtv0[r] if c == 0 else dtv1[r]
                prod = sft * 3072.0
                s_r = prod.astype(jnp.int32)
                s = s_r - (s_r.astype(jnp.float32) > prod).astype(jnp.int32)
                a = cv[c]
                b = cv[2 + c]

                def vec_body(i, _, r=r, c=c, a=a, b=b):
                    for u in range(4):
                        ii = i * 4 + u
                        v = bigin[r, c, pl.ds(ii * 16, 16)]
                        v = jnp.maximum(v * a + b, 0.0)
                        dbuf[pl.ds(ii * 16, 16)] = v
                        dbuf[pl.ds(ii * 16 + FEAT, 16)] = v
                    return 0

                lax.fori_loop(0, 48, vec_body, 0)

                def vec_body2(i, _, r=r, c=c, s=s):
                    for u in range(4):
                        ii = i * 4 + u
                        bigout[r, c, pl.ds(ii * 16, 16)] = dbuf[
                            pl.ds(FEAT - s + ii * 16, 16)]
                    return 0

                lax.fori_loop(0, 48, vec_body2, 0)
                ws = _WIN0 if c == 0 else _WIN1
                w0 = bigout[r, c, pl.ds(ws, 16)]
                w1 = bigout[r, c, pl.ds(ws + 16, 16)]
                w1 = jnp.where(lane < 4, w1, 0.0)
                wbuf[r, pl.ds(c * 32, 16)] = w0
                wbuf[r, pl.ds(c * 32 + 16, 16)] = w1
        pltpu.sync_copy(bigout, T.at[pl.ds(n8, GRP)])
        pltpu.sync_copy(wbuf, W.at[pl.ds(n8, GRP)])
        return 0

    lax.fori_loop(0, ngrp, grp_body, 0)


def _sc_mesh():
    return plsc.VectorSubcoreMesh(core_axis_name="c", subcore_axis_name="s",
                                  num_cores=2, num_subcores=16)


def _sc_roll(h2, dtp_pad, dts_pad, coef, interpret=False):
    mesh = _sc_mesh()
    return pl.kernel(
        _sc_roll_body,
        out_type=(
            jax.ShapeDtypeStruct((N_NODES, 2, FEAT), jnp.float32),
            jax.ShapeDtypeStruct((N_PAD, WINW), jnp.float32),
        ),
        mesh=mesh,
        scratch_types=[
            pltpu.VMEM((GRP, 2, FEAT), jnp.float32),
            pltpu.VMEM((GRP, 2, FEAT), jnp.float32),
            pltpu.VMEM((GRP, WINW), jnp.float32),
            pltpu.VMEM((2 * ROLL_BLK + 16,), jnp.float32),
            pltpu.VMEM((16,), jnp.float32),
            pltpu.VMEM((2 * FEAT,), jnp.float32),
        ],
        compiler_params=pltpu.CompilerParams(needs_layout_passes=False),
        interpret=interpret,
    )(h2, dtp_pad, dts_pad, coef)


def _sparse_part_jnp(h2, A2, B2, edge_index, dtp, dts):
    value = jnp.maximum(h2 * A2[None, :, None] + B2[None, :, None], 0.0)
    sp = (dtp * 3072.0).astype(jnp.int32)
    ss = (dts * 3072.0).astype(jnp.int32)
    L = FEAT
    idx0 = (jnp.arange(L)[None, :] - sp[:, None]) % L
    idx1 = (jnp.arange(L)[None, :] - ss[:, None]) % L
    ch0 = jnp.take_along_axis(value[:, 0, :], idx0, axis=1)
    ch1 = jnp.take_along_axis(value[:, 1, :], idx1, axis=1)

    def sim(xc, start, end):
        src, dst = edge_index[0], edge_index[1]
        n = xc.shape[0]
        x_j = jnp.take(xc, src, axis=0)
        x_i = jnp.take(xc, dst, axis=0)
        alpha = jnp.exp(-jnp.sum(jnp.abs(x_i[:, start:end] - x_j[:, start:end]), axis=-1))
        a = jnp.exp(alpha - 1.0)
        denom = jax.ops.segment_sum(a, dst, num_segments=n)
        a = a / (jnp.take(denom, dst, axis=0) + 1e-16)
        msg = a[:, None] * x_j
        agg_sum = jax.ops.segment_sum(msg, dst, num_segments=n)
        cnt = jax.ops.segment_sum(jnp.ones_like(a), dst, num_segments=n)
        agg = agg_sum / jnp.maximum(cnt, 1.0)[:, None]
        return agg + xc

    out0 = sim(ch0, 990, 1010)
    out1 = sim(ch1, 1490, 1510)
    return jnp.stack([out0, out1], axis=1)


E_EDGES = 26000
E_PAD = 26624
EDGE_T = E_PAD // 16
SENT = 10008
STATS_N = 10368
HALF = N_NODES // 2
CHUNK = 288
NCHUNK = 18
FLUSH_T = CHUNK // 16
WBLK = 16


def _sc_sparse_body(T, W, srcs, dsts, rb, eb, out,
                    sib, dib, wsb, wdb, tmp0, tmp1, rows, acc, trow,
                    rbv, ebv, sem0, sem1):
    cid = lax.axis_index("c")
    sid = lax.axis_index("s")
    w = cid * 16 + sid
    lane = lax.iota(jnp.int32, 16)
    zf16 = jnp.zeros((16,), jnp.float32)

    pltpu.sync_copy(rb, rbv)
    pltpu.sync_copy(eb, ebv)
    rbs = rbv[pl.ds(w, 16)]
    ebs = ebv[pl.ds(w, 16)]
    lo_r = rbs[0]
    hi_r = rbs[1]
    e_start = ebs[0]
    e_end = ebs[1]

    def za(i, _):
        acc[0, pl.ds(i * 16, 16)] = zf16
        acc[1, pl.ds(i * 16, 16)] = zf16
        return 0
    lax.fori_loop(0, FEAT // 16, za, 0)

    def flush_row(cur, nxt, den0, den1, ct):
        ctv = jnp.maximum(jnp.full((16,), ct, jnp.float32), 1.0)
        s0 = (1.0 / ((jnp.full((16,), den0, jnp.float32) + 1e-16) * ctv))[0]
        s1 = (1.0 / ((jnp.full((16,), den1, jnp.float32) + 1e-16) * ctv))[0]
        pltpu.sync_copy(T.at[cur], trow)

        def fb(i, _):
            for u in range(4):
                ii = i * 4 + u
                trow[0, pl.ds(ii * 16, 16)] = (
                    trow[0, pl.ds(ii * 16, 16)] + acc[0, pl.ds(ii * 16, 16)] * s0)
                trow[1, pl.ds(ii * 16, 16)] = (
                    trow[1, pl.ds(ii * 16, 16)] + acc[1, pl.ds(ii * 16, 16)] * s1)
                acc[0, pl.ds(ii * 16, 16)] = zf16
                acc[1, pl.ds(ii * 16, 16)] = zf16
            return 0
        lax.fori_loop(0, FEAT // 64, fb, 0)
        pltpu.sync_copy(trow, out.at[cur])

        def gap(g, _):
            pltpu.sync_copy(T.at[cur + 1 + g], out.at[cur + 1 + g])
            return 0
        lax.fori_loop(0, nxt - cur - 1, gap, 0)

    def issue(boff, p):
        cp = pltpu.make_async_copy(
            T.at[sib.at[pl.ds(boff, 8)]], rows.at[p], sem0 if p == 0 else sem1)
        cp.start()

    def wait(p):
        cp = pltpu.make_async_copy(
            T.at[sib.at[pl.ds(0, 8)]], rows.at[p], sem0 if p == 0 else sem1)
        cp.wait()

    sb_start = e_start // 256
    sb_end = (e_end + 255) // 256

    def super_blk(sb, carry):
        eb0 = sb * 256
        pltpu.sync_copy(srcs.at[pl.ds(eb0, 256)], sib)
        pltpu.sync_copy(dsts.at[pl.ds(eb0, 256)], dib)
        issue(0, 0)

        def blk(bp, carry):
            for p in range(2):
                b = bp * 2 + p
                ge0 = eb0 + b * 8
                wait(p)

                @pl.when(b + 1 < 32)
                def _(b=b, p=p):
                    issue((b + 1) * 8, 1 - p)

                active = (ge0 + 8 > e_start) & (ge0 < e_end)

                def process(carry, b=b, p=p, ge0=ge0):
                    cur, den0, den1, ct = carry
                    pltpu.sync_copy(W.at[sib.at[pl.ds(b * 8, 8)]], wsb)
                    pltpu.sync_copy(W.at[dib.at[pl.ds(b * 8, 8)]], wdb)
                    tmp0[pl.ds(0, 16)] = zf16
                    tmp1[pl.ds(0, 16)] = zf16
                    for r in range(8):
                        dva = (jnp.abs(wsb[r, pl.ds(0, 16)] - wdb[r, pl.ds(0, 16)])
                               + jnp.abs(wsb[r, pl.ds(16, 16)] - wdb[r, pl.ds(16, 16)]))
                        dvb = (jnp.abs(wsb[r, pl.ds(32, 16)] - wdb[r, pl.ds(32, 16)])
                               + jnp.abs(wsb[r, pl.ds(48, 16)] - wdb[r, pl.ds(48, 16)]))
                        rr = jnp.full((16,), r, jnp.int32)
                        plsc.addupdate_scatter(tmp0, [rr], dva)
                        plsc.addupdate_scatter(tmp1, [rr], dvb)
                    m = (((ge0 + lane) >= e_start) & ((ge0 + lane) < e_end)
                         & (lane < 8))
                    a0v = jnp.where(m, jnp.exp(jnp.exp(-tmp0[pl.ds(0, 16)]) - 1.0), 0.0)
                    a1v = jnp.where(m, jnp.exp(jnp.exp(-tmp1[pl.ds(0, 16)]) - 1.0), 0.0)
                    cv = jnp.where(m, 1.0, 0.0)
                    dv = jnp.clip(dib[pl.ds(b * 8, 16)], lo_r, hi_r - 1)
                    for r in range(8):
                        d_r = dv[r]
                        changed = d_r != cur

                        @pl.when(changed)
                        def _(cur=cur, d_r=d_r, den0=den0, den1=den1, ct=ct):
                            flush_row(cur, d_r, den0, den1, ct)

                        den0 = jnp.where(changed, 0.0, den0) + a0v[r]
                        den1 = jnp.where(changed, 0.0, den1) + a1v[r]
                        ct = jnp.where(changed, 0.0, ct) + cv[r]
                        cur = d_r
                        a0r = a0v[r]
                        a1r = a1v[r]

                        def fm(i, _, r=r, p=p, a0r=a0r, a1r=a1r):
                            for u in range(4):
                                ii = i * 4 + u
                                acc[0, pl.ds(ii * 16, 16)] = (
                                    acc[0, pl.ds(ii * 16, 16)]
                                    + rows[p, r, 0, pl.ds(ii * 16, 16)] * a0r)
                                acc[1, pl.ds(ii * 16, 16)] = (
                                    acc[1, pl.ds(ii * 16, 16)]
                                    + rows[p, r, 1, pl.ds(ii * 16, 16)] * a1r)
                            return 0
                        lax.fori_loop(0, FEAT // 64, fm, 0)
                    return (cur, den0, den1, ct)

                carry = lax.cond(active, process, lambda c: c, carry)
            return carry

        return lax.fori_loop(0, 16, blk, carry)

    carry = lax.fori_loop(
        sb_start, sb_end, super_blk,
        (lo_r, jnp.float32(0.0), jnp.float32(0.0), jnp.float32(0.0)))
    cur, den0, den1, ct = carry
    flush_row(cur, hi_r, den0, den1, ct)


def _sc_sparse(T, W, srcs, dsts, rb, eb):
    return pl.kernel(
        _sc_sparse_body,
        out_type=jax.ShapeDtypeStruct((N_NODES, 2, FEAT), jnp.float32),
        mesh=_sc_mesh(),
        scratch_types=[
            pltpu.VMEM((256,), jnp.int32),
            pltpu.VMEM((256,), jnp.int32),
            pltpu.VMEM((8, WINW), jnp.float32),
            pltpu.VMEM((8, WINW), jnp.float32),
            pltpu.VMEM((16,), jnp.float32),
            pltpu.VMEM((16,), jnp.float32),
            pltpu.VMEM((2, 8, 2, FEAT), jnp.float32),
            pltpu.VMEM((2, FEAT), jnp.float32),
            pltpu.VMEM((2, FEAT), jnp.float32),
            pltpu.VMEM((48,), jnp.int32),
            pltpu.VMEM((48,), jnp.int32),
            pltpu.SemaphoreType.DMA,
            pltpu.SemaphoreType.DMA,
        ],
        compiler_params=pltpu.CompilerParams(needs_layout_passes=False),
    )(T, W, srcs, dsts, rb, eb)


def _sim_tail_jnp(T, W, edge_index):
    src, dst = edge_index[0], edge_index[1]
    n = T.shape[0]
    wj = W[:N_NODES][src]
    wi = W[:N_NODES][dst]
    d = jnp.abs(wi - wj)
    a0 = jnp.exp(jnp.exp(-jnp.sum(d[:, 0:32], axis=1)) - 1.0)
    a1 = jnp.exp(jnp.exp(-jnp.sum(d[:, 32:64], axis=1)) - 1.0)
    cnt = jax.ops.segment_sum(jnp.ones_like(a0), dst, num_segments=n)
    out = []
    for c, a in ((0, a0), (1, a1)):
        denom = jax.ops.segment_sum(a, dst, num_segments=n)
        w = a / (denom[dst] + 1e-16)
        agg = jax.ops.segment_sum(w[:, None] * T[src, c, :], dst, num_segments=n)
        out.append(agg / jnp.maximum(cnt, 1.0)[:, None] + T[:, c, :])
    return jnp.stack(out, axis=1)


def kernel(x, edge_index, dtp, dts, conv1_w, conv1_b, bn1_g, bn1_b, conv2_w, conv2_b, bn2_g, bn2_b):
    h2, A2, B2 = _dense_part(x, conv1_w, conv1_b, bn1_g, bn1_b,
                             conv2_w, conv2_b, bn2_g, bn2_b)
    dtp_pad = jnp.pad(dtp, (0, N_PAD - N_NODES))
    dts_pad = jnp.pad(dts, (0, N_PAD - N_NODES))
    coef = jnp.concatenate([A2, B2, jnp.zeros((12,), jnp.float32)]).astype(jnp.float32)
    T, W = _sc_roll(h2, dtp_pad, dts_pad, coef)
    dstp = jnp.pad(edge_index[1], (0, E_PAD - E_EDGES), constant_values=SENT)
    srcp = jnp.pad(edge_index[0], (0, E_PAD - E_EDGES))
    order = jnp.argsort(dstp)
    srcs = srcp[order]
    dsts = dstp[order]
    rb = (((jnp.arange(33, dtype=jnp.int32) * N_NODES) // 32) // 4) * 4
    ebnd = jnp.searchsorted(dsts, rb).astype(jnp.int32)
    rb48 = jnp.pad(rb, (0, 15))
    eb48 = jnp.pad(ebnd, (0, 15))
    return _sc_sparse(T, W, srcs, dsts, rb48, eb48)


# speedup vs baseline: 1.2697x; 1.2697x over previous
import functools
import math

import jax
import jax.numpy as jnp
from jax import lax
from jax.experimental import pallas as pl
from jax.experimental.pallas import tpu as pltpu
from jax.experimental.pallas import tpu_sc as plsc

N_NODES = 10000
FEAT = 3072
NL = N_NODES * FEAT
TC_BLOCK = 40
TC_GRID = N_NODES // TC_BLOCK


def _conv2ch(x, w_ref, b_ref):
    B = x.shape[0]
    z = jnp.zeros((B, 3), dtype=jnp.float32)
    acc0 = jnp.full((B, FEAT), b_ref[0], dtype=jnp.float32)
    acc1 = jnp.full((B, FEAT), b_ref[1], dtype=jnp.float32)
    for i in range(2):
        xp = jnp.concatenate([z, x[:, i, :], z], axis=1)
        for k in range(7):
            s = xp[:, k:k + FEAT]
            acc0 = acc0 + w_ref[0, i, k] * s
            acc1 = acc1 + w_ref[1, i, k] * s
    return acc0, acc1


def _stats_update(i, h0, h1, st_ref, acc_ref):
    @pl.when(i == 0)
    def _():
        for j in range(4):
            acc_ref[j] = 0.0

    acc_ref[0] += jnp.sum(h0)
    acc_ref[1] += jnp.sum(h1)
    acc_ref[2] += jnp.sum(h0 * h0)
    acc_ref[3] += jnp.sum(h1 * h1)

    @pl.when(i == TC_GRID - 1)
    def _():
        for j in range(4):
            st_ref[j] = acc_ref[j]


def _p1_body(x_ref, w_ref, b_ref, h_ref, st_ref, acc_ref):
    i = pl.program_id(0)
    h0, h1 = _conv2ch(x_ref[...], w_ref, b_ref)
    h_ref[:, 0, :] = h0
    h_ref[:, 1, :] = h1
    _stats_update(i, h0, h1, st_ref, acc_ref)


def _p2_body(x_ref, w_ref, b_ref, A_ref, B_ref, h_ref, st_ref, acc_ref):
    i = pl.program_id(0)
    x = x_ref[...]
    g0 = jnp.maximum(x[:, 0, :] * A_ref[0] + B_ref[0], 0.0)
    g1 = jnp.maximum(x[:, 1, :] * A_ref[1] + B_ref[1], 0.0)
    g = jnp.stack([g0, g1], axis=1)
    h0, h1 = _conv2ch(g, w_ref, b_ref)
    h_ref[:, 0, :] = h0
    h_ref[:, 1, :] = h1
    _stats_update(i, h0, h1, st_ref, acc_ref)


def _conv_stats_pass(body, args, interpret=False):
    return pl.pallas_call(
        body,
        grid=(TC_GRID,),
        in_specs=[pl.BlockSpec((TC_BLOCK, 2, FEAT), lambda i: (i, 0, 0))]
        + [pl.BlockSpec(memory_space=pltpu.SMEM)] * (len(args) - 1),
        out_specs=[
            pl.BlockSpec((TC_BLOCK, 2, FEAT), lambda i: (i, 0, 0)),
            pl.BlockSpec(memory_space=pltpu.SMEM),
        ],
        out_shape=[
            jax.ShapeDtypeStruct((N_NODES, 2, FEAT), jnp.float32),
            jax.ShapeDtypeStruct((4,), jnp.float32),
        ],
        scratch_shapes=[pltpu.SMEM((4,), jnp.float32)],
        interpret=interpret,
    )(*args)


def _affine_from_stats(st, gamma, beta, eps=1e-5):
    mean = st[:2] / NL
    var = st[2:] / NL - mean * mean
    A = gamma / jnp.sqrt(var + eps)
    B = beta - mean * A
    return A, B


def _dense_part(x, conv1_w, conv1_b, bn1_g, bn1_b, conv2_w, conv2_b, bn2_g, bn2_b,
                interpret=False):
    h1, st1 = _conv_stats_pass(_p1_body, (x, conv1_w, conv1_b), interpret)
    A1, B1 = _affine_from_stats(st1, bn1_g, bn1_b)
    h2, st2 = _conv_stats_pass(_p2_body, (h1, conv2_w, conv2_b, A1, B1), interpret)
    A2, B2 = _affine_from_stats(st2, bn2_g, bn2_b)
    return h2, A2, B2


ROLL_BLK = 320
GRP = 8
N_PAD = 10240
WINW = 128
_WIN0 = 990
_WIN1 = 1490


def _sc_roll_body(h2, dtp, dts, coef, T, W, bigin, bigout,
                  wbuf, dtbuf, coefv, dbuf):
    cid = lax.axis_index("c")
    sid = lax.axis_index("s")
    wid = sid * 2 + cid
    base = wid * ROLL_BLK
    nrows = jnp.minimum(ROLL_BLK, N_NODES - base)
    ngrp = (nrows + GRP - 1) // GRP
    pltpu.sync_copy(dtp.at[pl.ds(base, ROLL_BLK)], dtbuf.at[pl.ds(0, ROLL_BLK)])
    pltpu.sync_copy(dts.at[pl.ds(base, ROLL_BLK)], dtbuf.at[pl.ds(ROLL_BLK, ROLL_BLK)])
    pltpu.sync_copy(coef, coefv)
    lane = lax.iota(jnp.int32, 16)
    cv = coefv[...]
    zf16 = jnp.zeros((16,), jnp.float32)
    for r0 in range(GRP):
        for k0 in range(WINW // 16):
            wbuf[r0, pl.ds(k0 * 16, 16)] = zf16

    def grp_body(g, _):
        n8 = base + g * GRP
        pltpu.sync_copy(h2.at[pl.ds(n8, GRP)], bigin)
        dtv0 = dtbuf[pl.ds(g * GRP, 16)]
        dtv1 = dtbuf[pl.ds(g * GRP + ROLL_BLK, 16)]
        for r in range(GRP):
            for c in range(2):
                sft = dtv0[r] if c == 0 else dtv1[r]
                prod = sft * 3072.0
                s_r = prod.astype(jnp.int32)
                s = s_r - (s_r.astype(jnp.float32) > prod).astype(jnp.int32)
                a = cv[c]
                b = cv[2 + c]

                def vec_body(i, _, r=r, c=c, a=a, b=b):
                    for u in range(8):
                        ii = i * 8 + u
                        v = bigin[r, c, pl.ds(ii * 16, 16)]
                        v = jnp.maximum(v * a + b, 0.0)
                        dbuf[pl.ds(ii * 16, 16)] = v
                        dbuf[pl.ds(ii * 16 + FEAT, 16)] = v
                    return 0

                lax.fori_loop(0, 24, vec_body, 0)

                def vec_body2(i, _, r=r, c=c, s=s):
                    for u in range(8):
                        ii = i * 8 + u
                        bigout[r, c, pl.ds(ii * 16, 16)] = dbuf[
                            pl.ds(FEAT - s + ii * 16, 16)]
                    return 0

                lax.fori_loop(0, 24, vec_body2, 0)
                ws = _WIN0 if c == 0 else _WIN1
                w0 = bigout[r, c, pl.ds(ws, 16)]
                w1 = bigout[r, c, pl.ds(ws + 16, 16)]
                w1 = jnp.where(lane < 4, w1, 0.0)
                wbuf[r, pl.ds(c * 32, 16)] = w0
                wbuf[r, pl.ds(c * 32 + 16, 16)] = w1
        pltpu.sync_copy(bigout, T.at[pl.ds(n8, GRP)])
        pltpu.sync_copy(wbuf, W.at[pl.ds(n8, GRP)])
        return 0

    lax.fori_loop(0, ngrp, grp_body, 0)


def _sc_mesh():
    return plsc.VectorSubcoreMesh(core_axis_name="c", subcore_axis_name="s",
                                  num_cores=2, num_subcores=16)


def _sc_roll(h2, dtp_pad, dts_pad, coef, interpret=False):
    mesh = _sc_mesh()
    return pl.kernel(
        _sc_roll_body,
        out_type=(
            jax.ShapeDtypeStruct((N_NODES, 2, FEAT), jnp.float32),
            jax.ShapeDtypeStruct((N_PAD, WINW), jnp.float32),
        ),
        mesh=mesh,
        scratch_types=[
            pltpu.VMEM((GRP, 2, FEAT), jnp.float32),
            pltpu.VMEM((GRP, 2, FEAT), jnp.float32),
            pltpu.VMEM((GRP, WINW), jnp.float32),
            pltpu.VMEM((2 * ROLL_BLK + 16,), jnp.float32),
            pltpu.VMEM((16,), jnp.float32),
            pltpu.VMEM((2 * FEAT,), jnp.float32),
        ],
        compiler_params=pltpu.CompilerParams(needs_layout_passes=False),
        interpret=interpret,
    )(h2, dtp_pad, dts_pad, coef)


def _sparse_part_jnp(h2, A2, B2, edge_index, dtp, dts):
    value = jnp.maximum(h2 * A2[None, :, None] + B2[None, :, None], 0.0)
    sp = (dtp * 3072.0).astype(jnp.int32)
    ss = (dts * 3072.0).astype(jnp.int32)
    L = FEAT
    idx0 = (jnp.arange(L)[None, :] - sp[:, None]) % L
    idx1 = (jnp.arange(L)[None, :] - ss[:, None]) % L
    ch0 = jnp.take_along_axis(value[:, 0, :], idx0, axis=1)
    ch1 = jnp.take_along_axis(value[:, 1, :], idx1, axis=1)

    def sim(xc, start, end):
        src, dst = edge_index[0], edge_index[1]
        n = xc.shape[0]
        x_j = jnp.take(xc, src, axis=0)
        x_i = jnp.take(xc, dst, axis=0)
        alpha = jnp.exp(-jnp.sum(jnp.abs(x_i[:, start:end] - x_j[:, start:end]), axis=-1))
        a = jnp.exp(alpha - 1.0)
        denom = jax.ops.segment_sum(a, dst, num_segments=n)
        a = a / (jnp.take(denom, dst, axis=0) + 1e-16)
        msg = a[:, None] * x_j
        agg_sum = jax.ops.segment_sum(msg, dst, num_segments=n)
        cnt = jax.ops.segment_sum(jnp.ones_like(a), dst, num_segments=n)
        agg = agg_sum / jnp.maximum(cnt, 1.0)[:, None]
        return agg + xc

    out0 = sim(ch0, 990, 1010)
    out1 = sim(ch1, 1490, 1510)
    return jnp.stack([out0, out1], axis=1)


E_EDGES = 26000
E_PAD = 26624
EDGE_T = E_PAD // 16
SENT = 10008
STATS_N = 10368
HALF = N_NODES // 2
CHUNK = 288
NCHUNK = 18
FLUSH_T = CHUNK // 16
WBLK = 16


def _sc_sparse_body(T, W, srcs, dsts, rb, eb, out,
                    sib, dib, wsb, wdb, tmp0, tmp1, rows, acc, trow,
                    rbv, ebv, sem0, sem1):
    cid = lax.axis_index("c")
    sid = lax.axis_index("s")
    w = cid * 16 + sid
    lane = lax.iota(jnp.int32, 16)
    zf16 = jnp.zeros((16,), jnp.float32)

    pltpu.sync_copy(rb, rbv)
    pltpu.sync_copy(eb, ebv)
    rbs = rbv[pl.ds(w, 16)]
    ebs = ebv[pl.ds(w, 16)]
    lo_r = rbs[0]
    hi_r = rbs[1]
    e_start = ebs[0]
    e_end = ebs[1]

    def za(i, _):
        acc[0, pl.ds(i * 16, 16)] = zf16
        acc[1, pl.ds(i * 16, 16)] = zf16
        return 0
    lax.fori_loop(0, FEAT // 16, za, 0)

    def flush_row(cur, nxt, den0, den1, ct):
        ctv = jnp.maximum(jnp.full((16,), ct, jnp.float32), 1.0)
        s0 = (1.0 / ((jnp.full((16,), den0, jnp.float32) + 1e-16) * ctv))[0]
        s1 = (1.0 / ((jnp.full((16,), den1, jnp.float32) + 1e-16) * ctv))[0]
        pltpu.sync_copy(T.at[cur], trow)

        def fb(i, _):
            for u in range(4):
                ii = i * 4 + u
                trow[0, pl.ds(ii * 16, 16)] = (
                    trow[0, pl.ds(ii * 16, 16)] + acc[0, pl.ds(ii * 16, 16)] * s0)
                trow[1, pl.ds(ii * 16, 16)] = (
                    trow[1, pl.ds(ii * 16, 16)] + acc[1, pl.ds(ii * 16, 16)] * s1)
                acc[0, pl.ds(ii * 16, 16)] = zf16
                acc[1, pl.ds(ii * 16, 16)] = zf16
            return 0
        lax.fori_loop(0, FEAT // 64, fb, 0)
        pltpu.sync_copy(trow, out.at[cur])

        def gap(g, _):
            pltpu.sync_copy(T.at[cur + 1 + g], out.at[cur + 1 + g])
            return 0
        lax.fori_loop(0, nxt - cur - 1, gap, 0)

    def issue(boff, p):
        cp = pltpu.make_async_copy(
            T.at[sib.at[pl.ds(boff, 8)]], rows.at[p], sem0 if p == 0 else sem1)
        cp.start()

    def wait(p):
        cp = pltpu.make_async_copy(
            T.at[sib.at[pl.ds(0, 8)]], rows.at[p], sem0 if p == 0 else sem1)
        cp.wait()

    sb_start = e_start // 256
    sb_end = (e_end + 255) // 256

    def super_blk(sb, carry):
        eb0 = sb * 256
        pltpu.sync_copy(srcs.at[pl.ds(eb0, 256)], sib)
        pltpu.sync_copy(dsts.at[pl.ds(eb0, 256)], dib)
        issue(0, 0)

        def blk(bp, carry):
            for p in range(2):
                b = bp * 2 + p
                ge0 = eb0 + b * 8
                wait(p)

                @pl.when(b + 1 < 32)
                def _(b=b, p=p):
                    issue((b + 1) * 8, 1 - p)

                active = (ge0 + 8 > e_start) & (ge0 < e_end)

                def process(carry, b=b, p=p, ge0=ge0):
                    cur, den0, den1, ct = carry
                    pltpu.sync_copy(W.at[sib.at[pl.ds(b * 8, 8)]], wsb)
                    pltpu.sync_copy(W.at[dib.at[pl.ds(b * 8, 8)]], wdb)
                    tmp0[pl.ds(0, 16)] = zf16
                    tmp1[pl.ds(0, 16)] = zf16
                    for r in range(8):
                        dva = (jnp.abs(wsb[r, pl.ds(0, 16)] - wdb[r, pl.ds(0, 16)])
                               + jnp.abs(wsb[r, pl.ds(16, 16)] - wdb[r, pl.ds(16, 16)]))
                        dvb = (jnp.abs(wsb[r, pl.ds(32, 16)] - wdb[r, pl.ds(32, 16)])
                               + jnp.abs(wsb[r, pl.ds(48, 16)] - wdb[r, pl.ds(48, 16)]))
                        rr = jnp.full((16,), r, jnp.int32)
                        plsc.addupdate_scatter(tmp0, [rr], dva)
                        plsc.addupdate_scatter(tmp1, [rr], dvb)
                    m = (((ge0 + lane) >= e_start) & ((ge0 + lane) < e_end)
                         & (lane < 8))
                    a0v = jnp.where(m, jnp.exp(jnp.exp(-tmp0[pl.ds(0, 16)]) - 1.0), 0.0)
                    a1v = jnp.where(m, jnp.exp(jnp.exp(-tmp1[pl.ds(0, 16)]) - 1.0), 0.0)
                    cv = jnp.where(m, 1.0, 0.0)
                    dv = jnp.clip(dib[pl.ds(b * 8, 16)], lo_r, hi_r - 1)
                    for r in range(8):
                        d_r = dv[r]
                        changed = d_r != cur

                        @pl.when(changed)
                        def _(cur=cur, d_r=d_r, den0=den0, den1=den1, ct=ct):
                            flush_row(cur, d_r, den0, den1, ct)

                        den0 = jnp.where(changed, 0.0, den0) + a0v[r]
                        den1 = jnp.where(changed, 0.0, den1) + a1v[r]
                        ct = jnp.where(changed, 0.0, ct) + cv[r]
                        cur = d_r
                        a0r = a0v[r]
                        a1r = a1v[r]

                        def fm(i, _, r=r, p=p, a0r=a0r, a1r=a1r):
                            for u in range(8):
                                ii = i * 8 + u
                                acc[0, pl.ds(ii * 16, 16)] = (
                                    acc[0, pl.ds(ii * 16, 16)]
                                    + rows[p, r, 0, pl.ds(ii * 16, 16)] * a0r)
                                acc[1, pl.ds(ii * 16, 16)] = (
                                    acc[1, pl.ds(ii * 16, 16)]
                                    + rows[p, r, 1, pl.ds(ii * 16, 16)] * a1r)
                            return 0
                        lax.fori_loop(0, FEAT // 128, fm, 0)
                    return (cur, den0, den1, ct)

                carry = lax.cond(active, process, lambda c: c, carry)
            return carry

        return lax.fori_loop(0, 16, blk, carry)

    carry = lax.fori_loop(
        sb_start, sb_end, super_blk,
        (lo_r, jnp.float32(0.0), jnp.float32(0.0), jnp.float32(0.0)))
    cur, den0, den1, ct = carry
    flush_row(cur, hi_r, den0, den1, ct)


def _sc_sparse(T, W, srcs, dsts, rb, eb):
    return pl.kernel(
        _sc_sparse_body,
        out_type=jax.ShapeDtypeStruct((N_NODES, 2, FEAT), jnp.float32),
        mesh=_sc_mesh(),
        scratch_types=[
            pltpu.VMEM((256,), jnp.int32),
            pltpu.VMEM((256,), jnp.int32),
            pltpu.VMEM((8, WINW), jnp.float32),
            pltpu.VMEM((8, WINW), jnp.float32),
            pltpu.VMEM((16,), jnp.float32),
            pltpu.VMEM((16,), jnp.float32),
            pltpu.VMEM((2, 8, 2, FEAT), jnp.float32),
            pltpu.VMEM((2, FEAT), jnp.float32),
            pltpu.VMEM((2, FEAT), jnp.float32),
            pltpu.VMEM((48,), jnp.int32),
            pltpu.VMEM((48,), jnp.int32),
            pltpu.SemaphoreType.DMA,
            pltpu.SemaphoreType.DMA,
        ],
        compiler_params=pltpu.CompilerParams(needs_layout_passes=False),
    )(T, W, srcs, dsts, rb, eb)


def _sim_tail_jnp(T, W, edge_index):
    src, dst = edge_index[0], edge_index[1]
    n = T.shape[0]
    wj = W[:N_NODES][src]
    wi = W[:N_NODES][dst]
    d = jnp.abs(wi - wj)
    a0 = jnp.exp(jnp.exp(-jnp.sum(d[:, 0:32], axis=1)) - 1.0)
    a1 = jnp.exp(jnp.exp(-jnp.sum(d[:, 32:64], axis=1)) - 1.0)
    cnt = jax.ops.segment_sum(jnp.ones_like(a0), dst, num_segments=n)
    out = []
    for c, a in ((0, a0), (1, a1)):
        denom = jax.ops.segment_sum(a, dst, num_segments=n)
        w = a / (denom[dst] + 1e-16)
        agg = jax.ops.segment_sum(w[:, None] * T[src, c, :], dst, num_segments=n)
        out.append(agg / jnp.maximum(cnt, 1.0)[:, None] + T[:, c, :])
    return jnp.stack(out, axis=1)


def kernel(x, edge_index, dtp, dts, conv1_w, conv1_b, bn1_g, bn1_b, conv2_w, conv2_b, bn2_g, bn2_b):
    h2, A2, B2 = _dense_part(x, conv1_w, conv1_b, bn1_g, bn1_b,
                             conv2_w, conv2_b, bn2_g, bn2_b)
    dtp_pad = jnp.pad(dtp, (0, N_PAD - N_NODES))
    dts_pad = jnp.pad(dts, (0, N_PAD - N_NODES))
    coef = jnp.concatenate([A2, B2, jnp.zeros((12,), jnp.float32)]).astype(jnp.float32)
    T, W = _sc_roll(h2, dtp_pad, dts_pad, coef)
    dstp = jnp.pad(edge_index[1], (0, E_PAD - E_EDGES), constant_values=SENT)
    srcp = jnp.pad(edge_index[0], (0, E_PAD - E_EDGES))
    order = jnp.argsort(dstp)
    srcs = srcp[order]
    dsts = dstp[order]
    rb = (((jnp.arange(33, dtype=jnp.int32) * N_NODES) // 32) // 4) * 4
    ebnd = jnp.searchsorted(dsts, rb).astype(jnp.int32)
    rb48 = jnp.pad(rb, (0, 15))
    eb48 = jnp.pad(ebnd, (0, 15))
    return _sc_sparse(T, W, srcs, dsts, rb48, eb48)


# speedup vs baseline: 1.4031x; 1.1051x over previous
import functools
import math

import jax
import jax.numpy as jnp
from jax import lax
from jax.experimental import pallas as pl
from jax.experimental.pallas import tpu as pltpu
from jax.experimental.pallas import tpu_sc as plsc

N_NODES = 10000
FEAT = 3072
NL = N_NODES * FEAT
TC_BLOCK = 40
TC_GRID = N_NODES // TC_BLOCK


def _conv2ch(x, w_ref, b_ref):
    B = x.shape[0]
    z = jnp.zeros((B, 3), dtype=jnp.float32)
    acc0 = jnp.full((B, FEAT), b_ref[0], dtype=jnp.float32)
    acc1 = jnp.full((B, FEAT), b_ref[1], dtype=jnp.float32)
    for i in range(2):
        xp = jnp.concatenate([z, x[:, i, :], z], axis=1)
        for k in range(7):
            s = xp[:, k:k + FEAT]
            acc0 = acc0 + w_ref[0, i, k] * s
            acc1 = acc1 + w_ref[1, i, k] * s
    return acc0, acc1


def _stats_update(i, h0, h1, st_ref, acc_ref):
    @pl.when(i == 0)
    def _():
        for j in range(4):
            acc_ref[j] = 0.0

    acc_ref[0] += jnp.sum(h0)
    acc_ref[1] += jnp.sum(h1)
    acc_ref[2] += jnp.sum(h0 * h0)
    acc_ref[3] += jnp.sum(h1 * h1)

    @pl.when(i == TC_GRID - 1)
    def _():
        for j in range(4):
            st_ref[j] = acc_ref[j]


def _p1_body(x_ref, w_ref, b_ref, h_ref, st_ref, acc_ref):
    i = pl.program_id(0)
    h0, h1 = _conv2ch(x_ref[...], w_ref, b_ref)
    h_ref[:, 0, :] = h0
    h_ref[:, 1, :] = h1
    _stats_update(i, h0, h1, st_ref, acc_ref)


def _p2_body(x_ref, w_ref, b_ref, A_ref, B_ref, h_ref, st_ref, acc_ref):
    i = pl.program_id(0)
    x = x_ref[...]
    g0 = jnp.maximum(x[:, 0, :] * A_ref[0] + B_ref[0], 0.0)
    g1 = jnp.maximum(x[:, 1, :] * A_ref[1] + B_ref[1], 0.0)
    g = jnp.stack([g0, g1], axis=1)
    h0, h1 = _conv2ch(g, w_ref, b_ref)
    h_ref[:, 0, :] = h0
    h_ref[:, 1, :] = h1
    _stats_update(i, h0, h1, st_ref, acc_ref)


def _conv_stats_pass(body, args, interpret=False):
    return pl.pallas_call(
        body,
        grid=(TC_GRID,),
        in_specs=[pl.BlockSpec((TC_BLOCK, 2, FEAT), lambda i: (i, 0, 0))]
        + [pl.BlockSpec(memory_space=pltpu.SMEM)] * (len(args) - 1),
        out_specs=[
            pl.BlockSpec((TC_BLOCK, 2, FEAT), lambda i: (i, 0, 0)),
            pl.BlockSpec(memory_space=pltpu.SMEM),
        ],
        out_shape=[
            jax.ShapeDtypeStruct((N_NODES, 2, FEAT), jnp.float32),
            jax.ShapeDtypeStruct((4,), jnp.float32),
        ],
        scratch_shapes=[pltpu.SMEM((4,), jnp.float32)],
        interpret=interpret,
    )(*args)


def _affine_from_stats(st, gamma, beta, eps=1e-5):
    mean = st[:2] / NL
    var = st[2:] / NL - mean * mean
    A = gamma / jnp.sqrt(var + eps)
    B = beta - mean * A
    return A, B


def _dense_part(x, conv1_w, conv1_b, bn1_g, bn1_b, conv2_w, conv2_b, bn2_g, bn2_b,
                interpret=False):
    h1, st1 = _conv_stats_pass(_p1_body, (x, conv1_w, conv1_b), interpret)
    A1, B1 = _affine_from_stats(st1, bn1_g, bn1_b)
    h2, st2 = _conv_stats_pass(_p2_body, (h1, conv2_w, conv2_b, A1, B1), interpret)
    A2, B2 = _affine_from_stats(st2, bn2_g, bn2_b)
    return h2, A2, B2


ROLL_BLK = 320
GRP = 8
N_PAD = 10240
WINW = 128
_WIN0 = 990
_WIN1 = 1490


def _sc_roll_body(h2, dtp, dts, coef, T, W, bigin, bigout,
                  wbuf, dtbuf, coefv, dbuf):
    cid = lax.axis_index("c")
    sid = lax.axis_index("s")
    wid = sid * 2 + cid
    base = wid * ROLL_BLK
    nrows = jnp.minimum(ROLL_BLK, N_NODES - base)
    ngrp = (nrows + GRP - 1) // GRP
    pltpu.sync_copy(dtp.at[pl.ds(base, ROLL_BLK)], dtbuf.at[pl.ds(0, ROLL_BLK)])
    pltpu.sync_copy(dts.at[pl.ds(base, ROLL_BLK)], dtbuf.at[pl.ds(ROLL_BLK, ROLL_BLK)])
    pltpu.sync_copy(coef, coefv)
    lane = lax.iota(jnp.int32, 16)
    cv = coefv[...]
    zf16 = jnp.zeros((16,), jnp.float32)
    for r0 in range(GRP):
        for k0 in range(WINW // 16):
            wbuf[r0, pl.ds(k0 * 16, 16)] = zf16

    def grp_body(g, _):
        n8 = base + g * GRP
        pltpu.sync_copy(h2.at[pl.ds(n8, GRP)], bigin)
        dtv0 = dtbuf[pl.ds(g * GRP, 16)]
        dtv1 = dtbuf[pl.ds(g * GRP + ROLL_BLK, 16)]
        for r in range(GRP):
            for c in range(2):
                sft = dtv0[r] if c == 0 else dtv1[r]
                prod = sft * 3072.0
                s_r = prod.astype(jnp.int32)
                s = s_r - (s_r.astype(jnp.float32) > prod).astype(jnp.int32)
                a = cv[c]
                b = cv[2 + c]

                def vec_body(i, _, r=r, c=c, a=a, b=b):
                    for u in range(16):
                        ii = i * 16 + u
                        v = bigin[r, c, pl.ds(ii * 16, 16)]
                        v = jnp.maximum(v * a + b, 0.0)
                        dbuf[pl.ds(ii * 16, 16)] = v
                        dbuf[pl.ds(ii * 16 + FEAT, 16)] = v
                    return 0

                lax.fori_loop(0, 12, vec_body, 0)

                def vec_body2(i, _, r=r, c=c, s=s):
                    for u in range(16):
                        ii = i * 16 + u
                        bigout[r, c, pl.ds(ii * 16, 16)] = dbuf[
                            pl.ds(FEAT - s + ii * 16, 16)]
                    return 0

                lax.fori_loop(0, 12, vec_body2, 0)
                ws = _WIN0 if c == 0 else _WIN1
                w0 = bigout[r, c, pl.ds(ws, 16)]
                w1 = bigout[r, c, pl.ds(ws + 16, 16)]
                w1 = jnp.where(lane < 4, w1, 0.0)
                wbuf[r, pl.ds(c * 32, 16)] = w0
                wbuf[r, pl.ds(c * 32 + 16, 16)] = w1
        pltpu.sync_copy(bigout, T.at[pl.ds(n8, GRP)])
        pltpu.sync_copy(wbuf, W.at[pl.ds(n8, GRP)])
        return 0

    lax.fori_loop(0, ngrp, grp_body, 0)


def _sc_mesh():
    return plsc.VectorSubcoreMesh(core_axis_name="c", subcore_axis_name="s",
                                  num_cores=2, num_subcores=16)


def _sc_roll(h2, dtp_pad, dts_pad, coef, interpret=False):
    mesh = _sc_mesh()
    return pl.kernel(
        _sc_roll_body,
        out_type=(
            jax.ShapeDtypeStruct((N_NODES, 2, FEAT), jnp.float32),
            jax.ShapeDtypeStruct((N_PAD, WINW), jnp.float32),
        ),
        mesh=mesh,
        scratch_types=[
            pltpu.VMEM((GRP, 2, FEAT), jnp.float32),
            pltpu.VMEM((GRP, 2, FEAT), jnp.float32),
            pltpu.VMEM((GRP, WINW), jnp.float32),
            pltpu.VMEM((2 * ROLL_BLK + 16,), jnp.float32),
            pltpu.VMEM((16,), jnp.float32),
            pltpu.VMEM((2 * FEAT,), jnp.float32),
        ],
        compiler_params=pltpu.CompilerParams(needs_layout_passes=False),
        interpret=interpret,
    )(h2, dtp_pad, dts_pad, coef)


def _sparse_part_jnp(h2, A2, B2, edge_index, dtp, dts):
    value = jnp.maximum(h2 * A2[None, :, None] + B2[None, :, None], 0.0)
    sp = (dtp * 3072.0).astype(jnp.int32)
    ss = (dts * 3072.0).astype(jnp.int32)
    L = FEAT
    idx0 = (jnp.arange(L)[None, :] - sp[:, None]) % L
    idx1 = (jnp.arange(L)[None, :] - ss[:, None]) % L
    ch0 = jnp.take_along_axis(value[:, 0, :], idx0, axis=1)
    ch1 = jnp.take_along_axis(value[:, 1, :], idx1, axis=1)

    def sim(xc, start, end):
        src, dst = edge_index[0], edge_index[1]
        n = xc.shape[0]
        x_j = jnp.take(xc, src, axis=0)
        x_i = jnp.take(xc, dst, axis=0)
        alpha = jnp.exp(-jnp.sum(jnp.abs(x_i[:, start:end] - x_j[:, start:end]), axis=-1))
        a = jnp.exp(alpha - 1.0)
        denom = jax.ops.segment_sum(a, dst, num_segments=n)
        a = a / (jnp.take(denom, dst, axis=0) + 1e-16)
        msg = a[:, None] * x_j
        agg_sum = jax.ops.segment_sum(msg, dst, num_segments=n)
        cnt = jax.ops.segment_sum(jnp.ones_like(a), dst, num_segments=n)
        agg = agg_sum / jnp.maximum(cnt, 1.0)[:, None]
        return agg + xc

    out0 = sim(ch0, 990, 1010)
    out1 = sim(ch1, 1490, 1510)
    return jnp.stack([out0, out1], axis=1)


E_EDGES = 26000
E_PAD = 26624
EDGE_T = E_PAD // 16
SENT = 10008
STATS_N = 10368
HALF = N_NODES // 2
CHUNK = 288
NCHUNK = 18
FLUSH_T = CHUNK // 16
WBLK = 16


def _sc_sparse_body(T, W, srcs, dsts, rb, eb, out,
                    sib, dib, wsb, wdb, tmp0, tmp1, rows, acc, trow,
                    rbv, ebv, sem0, sem1):
    cid = lax.axis_index("c")
    sid = lax.axis_index("s")
    w = cid * 16 + sid
    lane = lax.iota(jnp.int32, 16)
    zf16 = jnp.zeros((16,), jnp.float32)

    pltpu.sync_copy(rb, rbv)
    pltpu.sync_copy(eb, ebv)
    rbs = rbv[pl.ds(w, 16)]
    ebs = ebv[pl.ds(w, 16)]
    lo_r = rbs[0]
    hi_r = rbs[1]
    e_start = ebs[0]
    e_end = ebs[1]

    def za(i, _):
        acc[0, pl.ds(i * 16, 16)] = zf16
        acc[1, pl.ds(i * 16, 16)] = zf16
        return 0
    lax.fori_loop(0, FEAT // 16, za, 0)

    def flush_row(cur, nxt, den0, den1, ct):
        ctv = jnp.maximum(jnp.full((16,), ct, jnp.float32), 1.0)
        s0 = (1.0 / ((jnp.full((16,), den0, jnp.float32) + 1e-16) * ctv))[0]
        s1 = (1.0 / ((jnp.full((16,), den1, jnp.float32) + 1e-16) * ctv))[0]
        pltpu.sync_copy(T.at[cur], trow)

        def fb(i, _):
            for u in range(8):
                ii = i * 8 + u
                trow[0, pl.ds(ii * 16, 16)] = (
                    trow[0, pl.ds(ii * 16, 16)] + acc[0, pl.ds(ii * 16, 16)] * s0)
                trow[1, pl.ds(ii * 16, 16)] = (
                    trow[1, pl.ds(ii * 16, 16)] + acc[1, pl.ds(ii * 16, 16)] * s1)
                acc[0, pl.ds(ii * 16, 16)] = zf16
                acc[1, pl.ds(ii * 16, 16)] = zf16
            return 0
        lax.fori_loop(0, FEAT // 128, fb, 0)
        pltpu.sync_copy(trow, out.at[cur])

        def gap(g, _):
            pltpu.sync_copy(T.at[cur + 1 + g], out.at[cur + 1 + g])
            return 0
        lax.fori_loop(0, nxt - cur - 1, gap, 0)

    def issue(boff, p):
        cp = pltpu.make_async_copy(
            T.at[sib.at[pl.ds(boff, 8)]], rows.at[p], sem0 if p == 0 else sem1)
        cp.start()

    def wait(p):
        cp = pltpu.make_async_copy(
            T.at[sib.at[pl.ds(0, 8)]], rows.at[p], sem0 if p == 0 else sem1)
        cp.wait()

    sb_start = e_start // 256
    sb_end = (e_end + 255) // 256

    def super_blk(sb, carry):
        eb0 = sb * 256
        pltpu.sync_copy(srcs.at[pl.ds(eb0, 256)], sib)
        pltpu.sync_copy(dsts.at[pl.ds(eb0, 256)], dib)
        issue(0, 0)

        def blk(bp, carry):
            for p in range(2):
                b = bp * 2 + p
                ge0 = eb0 + b * 8
                wait(p)

                @pl.when(b + 1 < 32)
                def _(b=b, p=p):
                    issue((b + 1) * 8, 1 - p)

                active = (ge0 + 8 > e_start) & (ge0 < e_end)

                def process(carry, b=b, p=p, ge0=ge0):
                    cur, den0, den1, ct = carry
                    pltpu.sync_copy(W.at[sib.at[pl.ds(b * 8, 8)]], wsb)
                    pltpu.sync_copy(W.at[dib.at[pl.ds(b * 8, 8)]], wdb)
                    tmp0[pl.ds(0, 16)] = zf16
                    tmp1[pl.ds(0, 16)] = zf16
                    for r in range(8):
                        dva = (jnp.abs(wsb[r, pl.ds(0, 16)] - wdb[r, pl.ds(0, 16)])
                               + jnp.abs(wsb[r, pl.ds(16, 16)] - wdb[r, pl.ds(16, 16)]))
                        dvb = (jnp.abs(wsb[r, pl.ds(32, 16)] - wdb[r, pl.ds(32, 16)])
                               + jnp.abs(wsb[r, pl.ds(48, 16)] - wdb[r, pl.ds(48, 16)]))
                        rr = jnp.full((16,), r, jnp.int32)
                        plsc.addupdate_scatter(tmp0, [rr], dva)
                        plsc.addupdate_scatter(tmp1, [rr], dvb)
                    m = (((ge0 + lane) >= e_start) & ((ge0 + lane) < e_end)
                         & (lane < 8))
                    a0v = jnp.where(m, jnp.exp(jnp.exp(-tmp0[pl.ds(0, 16)]) - 1.0), 0.0)
                    a1v = jnp.where(m, jnp.exp(jnp.exp(-tmp1[pl.ds(0, 16)]) - 1.0), 0.0)
                    cv = jnp.where(m, 1.0, 0.0)
                    dv = jnp.clip(dib[pl.ds(b * 8, 16)], lo_r, hi_r - 1)
                    for r in range(8):
                        d_r = dv[r]
                        changed = d_r != cur

                        @pl.when(changed)
                        def _(cur=cur, d_r=d_r, den0=den0, den1=den1, ct=ct):
                            flush_row(cur, d_r, den0, den1, ct)

                        den0 = jnp.where(changed, 0.0, den0) + a0v[r]
                        den1 = jnp.where(changed, 0.0, den1) + a1v[r]
                        ct = jnp.where(changed, 0.0, ct) + cv[r]
                        cur = d_r
                        a0r = a0v[r]
                        a1r = a1v[r]

                        def fm(i, _, r=r, p=p, a0r=a0r, a1r=a1r):
                            for u in range(16):
                                ii = i * 16 + u
                                acc[0, pl.ds(ii * 16, 16)] = (
                                    acc[0, pl.ds(ii * 16, 16)]
                                    + rows[p, r, 0, pl.ds(ii * 16, 16)] * a0r)
                                acc[1, pl.ds(ii * 16, 16)] = (
                                    acc[1, pl.ds(ii * 16, 16)]
                                    + rows[p, r, 1, pl.ds(ii * 16, 16)] * a1r)
                            return 0
                        lax.fori_loop(0, FEAT // 256, fm, 0)
                    return (cur, den0, den1, ct)

                carry = lax.cond(active, process, lambda c: c, carry)
            return carry

        return lax.fori_loop(0, 16, blk, carry)

    carry = lax.fori_loop(
        sb_start, sb_end, super_blk,
        (lo_r, jnp.float32(0.0), jnp.float32(0.0), jnp.float32(0.0)))
    cur, den0, den1, ct = carry
    flush_row(cur, hi_r, den0, den1, ct)


def _sc_sparse(T, W, srcs, dsts, rb, eb):
    return pl.kernel(
        _sc_sparse_body,
        out_type=jax.ShapeDtypeStruct((N_NODES, 2, FEAT), jnp.float32),
        mesh=_sc_mesh(),
        scratch_types=[
            pltpu.VMEM((256,), jnp.int32),
            pltpu.VMEM((256,), jnp.int32),
            pltpu.VMEM((8, WINW), jnp.float32),
            pltpu.VMEM((8, WINW), jnp.float32),
            pltpu.VMEM((16,), jnp.float32),
            pltpu.VMEM((16,), jnp.float32),
            pltpu.VMEM((2, 8, 2, FEAT), jnp.float32),
            pltpu.VMEM((2, FEAT), jnp.float32),
            pltpu.VMEM((2, FEAT), jnp.float32),
            pltpu.VMEM((48,), jnp.int32),
            pltpu.VMEM((48,), jnp.int32),
            pltpu.SemaphoreType.DMA,
            pltpu.SemaphoreType.DMA,
        ],
        compiler_params=pltpu.CompilerParams(needs_layout_passes=False),
    )(T, W, srcs, dsts, rb, eb)


def _sim_tail_jnp(T, W, edge_index):
    src, dst = edge_index[0], edge_index[1]
    n = T.shape[0]
    wj = W[:N_NODES][src]
    wi = W[:N_NODES][dst]
    d = jnp.abs(wi - wj)
    a0 = jnp.exp(jnp.exp(-jnp.sum(d[:, 0:32], axis=1)) - 1.0)
    a1 = jnp.exp(jnp.exp(-jnp.sum(d[:, 32:64], axis=1)) - 1.0)
    cnt = jax.ops.segment_sum(jnp.ones_like(a0), dst, num_segments=n)
    out = []
    for c, a in ((0, a0), (1, a1)):
        denom = jax.ops.segment_sum(a, dst, num_segments=n)
        w = a / (denom[dst] + 1e-16)
        agg = jax.ops.segment_sum(w[:, None] * T[src, c, :], dst, num_segments=n)
        out.append(agg / jnp.maximum(cnt, 1.0)[:, None] + T[:, c, :])
    return jnp.stack(out, axis=1)


def kernel(x, edge_index, dtp, dts, conv1_w, conv1_b, bn1_g, bn1_b, conv2_w, conv2_b, bn2_g, bn2_b):
    h2, A2, B2 = _dense_part(x, conv1_w, conv1_b, bn1_g, bn1_b,
                             conv2_w, conv2_b, bn2_g, bn2_b)
    dtp_pad = jnp.pad(dtp, (0, N_PAD - N_NODES))
    dts_pad = jnp.pad(dts, (0, N_PAD - N_NODES))
    coef = jnp.concatenate([A2, B2, jnp.zeros((12,), jnp.float32)]).astype(jnp.float32)
    T, W = _sc_roll(h2, dtp_pad, dts_pad, coef)
    dstp = jnp.pad(edge_index[1], (0, E_PAD - E_EDGES), constant_values=SENT)
    srcp = jnp.pad(edge_index[0], (0, E_PAD - E_EDGES))
    order = jnp.argsort(dstp)
    srcs = srcp[order]
    dsts = dstp[order]
    rb = (((jnp.arange(33, dtype=jnp.int32) * N_NODES) // 32) // 4) * 4
    ebnd = jnp.searchsorted(dsts, rb).astype(jnp.int32)
    rb48 = jnp.pad(rb, (0, 15))
    eb48 = jnp.pad(ebnd, (0, 15))
    return _sc_sparse(T, W, srcs, dsts, rb48, eb48)


# speedup vs baseline: 1.4035x; 1.0003x over previous
import functools
import math

import jax
import jax.numpy as jnp
from jax import lax
from jax.experimental import pallas as pl
from jax.experimental.pallas import tpu as pltpu
from jax.experimental.pallas import tpu_sc as plsc

N_NODES = 10000
FEAT = 3072
NL = N_NODES * FEAT
TC_BLOCK = 40
TC_GRID = N_NODES // TC_BLOCK


def _conv2ch(x, w_ref, b_ref):
    B = x.shape[0]
    z = jnp.zeros((B, 3), dtype=jnp.float32)
    acc0 = jnp.full((B, FEAT), b_ref[0], dtype=jnp.float32)
    acc1 = jnp.full((B, FEAT), b_ref[1], dtype=jnp.float32)
    for i in range(2):
        xp = jnp.concatenate([z, x[:, i, :], z], axis=1)
        for k in range(7):
            s = xp[:, k:k + FEAT]
            acc0 = acc0 + w_ref[0, i, k] * s
            acc1 = acc1 + w_ref[1, i, k] * s
    return acc0, acc1


def _stats_update(i, h0, h1, st_ref, acc_ref):
    @pl.when(i == 0)
    def _():
        for j in range(4):
            acc_ref[j] = 0.0

    acc_ref[0] += jnp.sum(h0)
    acc_ref[1] += jnp.sum(h1)
    acc_ref[2] += jnp.sum(h0 * h0)
    acc_ref[3] += jnp.sum(h1 * h1)

    @pl.when(i == TC_GRID - 1)
    def _():
        for j in range(4):
            st_ref[j] = acc_ref[j]


def _p1_body(x_ref, w_ref, b_ref, h_ref, st_ref, acc_ref):
    i = pl.program_id(0)
    h0, h1 = _conv2ch(x_ref[...], w_ref, b_ref)
    h_ref[:, 0, :] = h0
    h_ref[:, 1, :] = h1
    _stats_update(i, h0, h1, st_ref, acc_ref)


def _p2_body(x_ref, w_ref, b_ref, A_ref, B_ref, h_ref, st_ref, acc_ref):
    i = pl.program_id(0)
    x = x_ref[...]
    g0 = jnp.maximum(x[:, 0, :] * A_ref[0] + B_ref[0], 0.0)
    g1 = jnp.maximum(x[:, 1, :] * A_ref[1] + B_ref[1], 0.0)
    g = jnp.stack([g0, g1], axis=1)
    h0, h1 = _conv2ch(g, w_ref, b_ref)
    h_ref[:, 0, :] = h0
    h_ref[:, 1, :] = h1
    _stats_update(i, h0, h1, st_ref, acc_ref)


def _conv_stats_pass(body, args, interpret=False):
    return pl.pallas_call(
        body,
        grid=(TC_GRID,),
        in_specs=[pl.BlockSpec((TC_BLOCK, 2, FEAT), lambda i: (i, 0, 0))]
        + [pl.BlockSpec(memory_space=pltpu.SMEM)] * (len(args) - 1),
        out_specs=[
            pl.BlockSpec((TC_BLOCK, 2, FEAT), lambda i: (i, 0, 0)),
            pl.BlockSpec(memory_space=pltpu.SMEM),
        ],
        out_shape=[
            jax.ShapeDtypeStruct((N_NODES, 2, FEAT), jnp.float32),
            jax.ShapeDtypeStruct((4,), jnp.float32),
        ],
        scratch_shapes=[pltpu.SMEM((4,), jnp.float32)],
        interpret=interpret,
    )(*args)


def _affine_from_stats(st, gamma, beta, eps=1e-5):
    mean = st[:2] / NL
    var = st[2:] / NL - mean * mean
    A = gamma / jnp.sqrt(var + eps)
    B = beta - mean * A
    return A, B


def _dense_part(x, conv1_w, conv1_b, bn1_g, bn1_b, conv2_w, conv2_b, bn2_g, bn2_b,
                interpret=False):
    h1, st1 = _conv_stats_pass(_p1_body, (x, conv1_w, conv1_b), interpret)
    A1, B1 = _affine_from_stats(st1, bn1_g, bn1_b)
    h2, st2 = _conv_stats_pass(_p2_body, (h1, conv2_w, conv2_b, A1, B1), interpret)
    A2, B2 = _affine_from_stats(st2, bn2_g, bn2_b)
    return h2, A2, B2


ROLL_BLK = 320
GRP = 8
N_PAD = 10240
WINW = 128
_WIN0 = 990
_WIN1 = 1490


def _sc_roll_body(h2, dtp, dts, coef, T, W, bigin, bigout,
                  wbuf, dtbuf, coefv, dbuf):
    cid = lax.axis_index("c")
    sid = lax.axis_index("s")
    wid = sid * 2 + cid
    base = wid * ROLL_BLK
    nrows = jnp.minimum(ROLL_BLK, N_NODES - base)
    ngrp = (nrows + GRP - 1) // GRP
    pltpu.sync_copy(dtp.at[pl.ds(base, ROLL_BLK)], dtbuf.at[pl.ds(0, ROLL_BLK)])
    pltpu.sync_copy(dts.at[pl.ds(base, ROLL_BLK)], dtbuf.at[pl.ds(ROLL_BLK, ROLL_BLK)])
    pltpu.sync_copy(coef, coefv)
    lane = lax.iota(jnp.int32, 16)
    cv = coefv[...]
    zf16 = jnp.zeros((16,), jnp.float32)
    for r0 in range(GRP):
        for k0 in range(WINW // 16):
            wbuf[r0, pl.ds(k0 * 16, 16)] = zf16

    def grp_body(g, _):
        n8 = base + g * GRP
        pltpu.sync_copy(h2.at[pl.ds(n8, GRP)], bigin)
        dtv0 = dtbuf[pl.ds(g * GRP, 16)]
        dtv1 = dtbuf[pl.ds(g * GRP + ROLL_BLK, 16)]
        for r in range(GRP):
            for c in range(2):
                sft = dtv0[r] if c == 0 else dtv1[r]
                prod = sft * 3072.0
                s_r = prod.astype(jnp.int32)
                s = s_r - (s_r.astype(jnp.float32) > prod).astype(jnp.int32)
                a = cv[c]
                b = cv[2 + c]

                def vec_body(i, _, r=r, c=c, a=a, b=b):
                    for u in range(16):
                        ii = i * 16 + u
                        v = bigin[r, c, pl.ds(ii * 16, 16)]
                        v = jnp.maximum(v * a + b, 0.0)
                        dbuf[pl.ds(ii * 16, 16)] = v
                        dbuf[pl.ds(ii * 16 + FEAT, 16)] = v
                    return 0

                lax.fori_loop(0, 12, vec_body, 0)

                def vec_body2(i, _, r=r, c=c, s=s):
                    for u in range(16):
                        ii = i * 16 + u
                        bigout[r, c, pl.ds(ii * 16, 16)] = dbuf[
                            pl.ds(FEAT - s + ii * 16, 16)]
                    return 0

                lax.fori_loop(0, 12, vec_body2, 0)
                ws = _WIN0 if c == 0 else _WIN1
                w0 = bigout[r, c, pl.ds(ws, 16)]
                w1 = bigout[r, c, pl.ds(ws + 16, 16)]
                w1 = jnp.where(lane < 4, w1, 0.0)
                wbuf[r, pl.ds(c * 32, 16)] = w0
                wbuf[r, pl.ds(c * 32 + 16, 16)] = w1
        pltpu.sync_copy(bigout, T.at[pl.ds(n8, GRP)])
        pltpu.sync_copy(wbuf, W.at[pl.ds(n8, GRP)])
        return 0

    lax.fori_loop(0, ngrp, grp_body, 0)


def _sc_mesh():
    return plsc.VectorSubcoreMesh(core_axis_name="c", subcore_axis_name="s",
                                  num_cores=2, num_subcores=16)


def _sc_roll(h2, dtp_pad, dts_pad, coef, interpret=False):
    mesh = _sc_mesh()
    return pl.kernel(
        _sc_roll_body,
        out_type=(
            jax.ShapeDtypeStruct((N_NODES, 2, FEAT), jnp.float32),
            jax.ShapeDtypeStruct((N_PAD, WINW), jnp.float32),
        ),
        mesh=mesh,
        scratch_types=[
            pltpu.VMEM((GRP, 2, FEAT), jnp.float32),
            pltpu.VMEM((GRP, 2, FEAT), jnp.float32),
            pltpu.VMEM((GRP, WINW), jnp.float32),
            pltpu.VMEM((2 * ROLL_BLK + 16,), jnp.float32),
            pltpu.VMEM((16,), jnp.float32),
            pltpu.VMEM((2 * FEAT,), jnp.float32),
        ],
        compiler_params=pltpu.CompilerParams(needs_layout_passes=False),
        interpret=interpret,
    )(h2, dtp_pad, dts_pad, coef)


E_EDGES = 26000
E_PAD = 26624
SENT = 10008


def _sc_sparse_body(T, W, srcs, dsts, rb, eb, out,
                    sib, dib, wsb, wdb, tmp0, tmp1, rows, acc, trow,
                    rbv, ebv, sem0, sem1):
    cid = lax.axis_index("c")
    sid = lax.axis_index("s")
    w = cid * 16 + sid
    lane = lax.iota(jnp.int32, 16)
    zf16 = jnp.zeros((16,), jnp.float32)

    pltpu.sync_copy(rb, rbv)
    pltpu.sync_copy(eb, ebv)
    rbs = rbv[pl.ds(w, 16)]
    ebs = ebv[pl.ds(w, 16)]
    lo_r = rbs[0]
    hi_r = rbs[1]
    e_start = ebs[0]
    e_end = ebs[1]

    def za(i, _):
        acc[0, pl.ds(i * 16, 16)] = zf16
        acc[1, pl.ds(i * 16, 16)] = zf16
        return 0
    lax.fori_loop(0, FEAT // 16, za, 0)

    def flush_row(cur, nxt, den0, den1, ct):
        ctv = jnp.maximum(jnp.full((16,), ct, jnp.float32), 1.0)
        s0 = (1.0 / ((jnp.full((16,), den0, jnp.float32) + 1e-16) * ctv))[0]
        s1 = (1.0 / ((jnp.full((16,), den1, jnp.float32) + 1e-16) * ctv))[0]
        pltpu.sync_copy(T.at[cur], trow)

        def fb(i, _):
            for u in range(8):
                ii = i * 8 + u
                trow[0, pl.ds(ii * 16, 16)] = (
                    trow[0, pl.ds(ii * 16, 16)] + acc[0, pl.ds(ii * 16, 16)] * s0)
                trow[1, pl.ds(ii * 16, 16)] = (
                    trow[1, pl.ds(ii * 16, 16)] + acc[1, pl.ds(ii * 16, 16)] * s1)
                acc[0, pl.ds(ii * 16, 16)] = zf16
                acc[1, pl.ds(ii * 16, 16)] = zf16
            return 0
        lax.fori_loop(0, FEAT // 128, fb, 0)
        pltpu.sync_copy(trow, out.at[cur])

        def gap(g, _):
            pltpu.sync_copy(T.at[cur + 1 + g], out.at[cur + 1 + g])
            return 0
        lax.fori_loop(0, nxt - cur - 1, gap, 0)

    def issue(boff, p):
        cp = pltpu.make_async_copy(
            T.at[sib.at[pl.ds(boff, 8)]], rows.at[p], sem0 if p == 0 else sem1)
        cp.start()

    def wait(p):
        cp = pltpu.make_async_copy(
            T.at[sib.at[pl.ds(0, 8)]], rows.at[p], sem0 if p == 0 else sem1)
        cp.wait()

    sb_start = e_start // 256
    sb_end = (e_end + 255) // 256

    def super_blk(sb, carry):
        eb0 = sb * 256
        pltpu.sync_copy(srcs.at[pl.ds(eb0, 256)], sib)
        pltpu.sync_copy(dsts.at[pl.ds(eb0, 256)], dib)
        issue(0, 0)

        def blk(bp, carry):
            for p in range(2):
                b = bp * 2 + p
                ge0 = eb0 + b * 8
                wait(p)

                @pl.when(b + 1 < 32)
                def _(b=b, p=p):
                    issue((b + 1) * 8, 1 - p)

                active = (ge0 + 8 > e_start) & (ge0 < e_end)

                def process(carry, b=b, p=p, ge0=ge0):
                    cur, den0, den1, ct = carry
                    pltpu.sync_copy(W.at[sib.at[pl.ds(b * 8, 8)]], wsb)
                    pltpu.sync_copy(W.at[dib.at[pl.ds(b * 8, 8)]], wdb)
                    tmp0[pl.ds(0, 16)] = zf16
                    tmp1[pl.ds(0, 16)] = zf16
                    for r in range(8):
                        dva = (jnp.abs(wsb[r, pl.ds(0, 16)] - wdb[r, pl.ds(0, 16)])
                               + jnp.abs(wsb[r, pl.ds(16, 16)] - wdb[r, pl.ds(16, 16)]))
                        dvb = (jnp.abs(wsb[r, pl.ds(32, 16)] - wdb[r, pl.ds(32, 16)])
                               + jnp.abs(wsb[r, pl.ds(48, 16)] - wdb[r, pl.ds(48, 16)]))
                        rr = jnp.full((16,), r, jnp.int32)
                        plsc.addupdate_scatter(tmp0, [rr], dva)
                        plsc.addupdate_scatter(tmp1, [rr], dvb)
                    m = (((ge0 + lane) >= e_start) & ((ge0 + lane) < e_end)
                         & (lane < 8))
                    a0v = jnp.where(m, jnp.exp(jnp.exp(-tmp0[pl.ds(0, 16)]) - 1.0), 0.0)
                    a1v = jnp.where(m, jnp.exp(jnp.exp(-tmp1[pl.ds(0, 16)]) - 1.0), 0.0)
                    cv = jnp.where(m, 1.0, 0.0)
                    dv = jnp.clip(dib[pl.ds(b * 8, 16)], lo_r, hi_r - 1)
                    for r in range(8):
                        d_r = dv[r]
                        changed = d_r != cur

                        @pl.when(changed)
                        def _(cur=cur, d_r=d_r, den0=den0, den1=den1, ct=ct):
                            flush_row(cur, d_r, den0, den1, ct)

                        den0 = jnp.where(changed, 0.0, den0) + a0v[r]
                        den1 = jnp.where(changed, 0.0, den1) + a1v[r]
                        ct = jnp.where(changed, 0.0, ct) + cv[r]
                        cur = d_r
                        a0r = a0v[r]
                        a1r = a1v[r]

                        def fm(i, _, r=r, p=p, a0r=a0r, a1r=a1r):
                            for u in range(16):
                                ii = i * 16 + u
                                acc[0, pl.ds(ii * 16, 16)] = (
                                    acc[0, pl.ds(ii * 16, 16)]
                                    + rows[p, r, 0, pl.ds(ii * 16, 16)] * a0r)
                                acc[1, pl.ds(ii * 16, 16)] = (
                                    acc[1, pl.ds(ii * 16, 16)]
                                    + rows[p, r, 1, pl.ds(ii * 16, 16)] * a1r)
                            return 0
                        lax.fori_loop(0, FEAT // 256, fm, 0)
                    return (cur, den0, den1, ct)

                carry = lax.cond(active, process, lambda c: c, carry)
            return carry

        return lax.fori_loop(0, 16, blk, carry)

    carry = lax.fori_loop(
        sb_start, sb_end, super_blk,
        (lo_r, jnp.float32(0.0), jnp.float32(0.0), jnp.float32(0.0)))
    cur, den0, den1, ct = carry
    flush_row(cur, hi_r, den0, den1, ct)


def _sc_sparse(T, W, srcs, dsts, rb, eb):
    return pl.kernel(
        _sc_sparse_body,
        out_type=jax.ShapeDtypeStruct((N_NODES, 2, FEAT), jnp.float32),
        mesh=_sc_mesh(),
        scratch_types=[
            pltpu.VMEM((256,), jnp.int32),
            pltpu.VMEM((256,), jnp.int32),
            pltpu.VMEM((8, WINW), jnp.float32),
            pltpu.VMEM((8, WINW), jnp.float32),
            pltpu.VMEM((16,), jnp.float32),
            pltpu.VMEM((16,), jnp.float32),
            pltpu.VMEM((2, 8, 2, FEAT), jnp.float32),
            pltpu.VMEM((2, FEAT), jnp.float32),
            pltpu.VMEM((2, FEAT), jnp.float32),
            pltpu.VMEM((48,), jnp.int32),
            pltpu.VMEM((48,), jnp.int32),
            pltpu.SemaphoreType.DMA,
            pltpu.SemaphoreType.DMA,
        ],
        compiler_params=pltpu.CompilerParams(needs_layout_passes=False),
    )(T, W, srcs, dsts, rb, eb)


def kernel(x, edge_index, dtp, dts, conv1_w, conv1_b, bn1_g, bn1_b, conv2_w, conv2_b, bn2_g, bn2_b):
    h2, A2, B2 = _dense_part(x, conv1_w, conv1_b, bn1_g, bn1_b,
                             conv2_w, conv2_b, bn2_g, bn2_b)
    dtp_pad = jnp.pad(dtp, (0, N_PAD - N_NODES))
    dts_pad = jnp.pad(dts, (0, N_PAD - N_NODES))
    coef = jnp.concatenate([A2, B2, jnp.zeros((12,), jnp.float32)]).astype(jnp.float32)
    T, W = _sc_roll(h2, dtp_pad, dts_pad, coef)
    dstp = jnp.pad(edge_index[1], (0, E_PAD - E_EDGES), constant_values=SENT)
    srcp = jnp.pad(edge_index[0], (0, E_PAD - E_EDGES))
    order = jnp.argsort(dstp)
    srcs = srcp[order]
    dsts = dstp[order]
    rb = (((jnp.arange(33, dtype=jnp.int32) * N_NODES) // 32) // 4) * 4
    ebnd = jnp.searchsorted(dsts, rb).astype(jnp.int32)
    rb48 = jnp.pad(rb, (0, 15))
    eb48 = jnp.pad(ebnd, (0, 15))
    return _sc_sparse(T, W, srcs, dsts, rb48, eb48)
